# baseline (speedup 1.0000x reference)
import functools

import jax
import jax.numpy as jnp
from jax.experimental import pallas as pl
from jax.experimental.pallas import tpu as pltpu


def _se_block_kernel(x_ref, w1t_ref, b1t_ref, w2t_ref, b2t_ref, o_ref, *,
                     nb, inv_hw):
    w1t = w1t_ref[...]
    w2t = w2t_ref[...]
    b1t = b1t_ref[...]
    b2t = b2t_ref[...]
    for i in range(nb):
        x = x_ref[i]
        g = jax.lax.dot_general(
            w1t, x,
            dimension_numbers=(((1,), (0,)), ((), ())),
            preferred_element_type=jnp.float32,
        )
        h = jnp.sum(g, axis=1, keepdims=True) * inv_hw + b1t
        h = jnp.maximum(h, 0.0)
        z = jax.lax.dot_general(
            w2t, h,
            dimension_numbers=(((1,), (0,)), ((), ())),
            preferred_element_type=jnp.float32,
        ) + b2t
        s = jax.nn.sigmoid(z)
        o_ref[i] = x * s.astype(x.dtype)


def kernel(x, w1, b1, w2, b2):
    B, C, H, W = x.shape
    HW = H * W
    hidden = w1.shape[1]
    x3d = x.reshape(B, C, HW)

    slab_bytes = C * HW * x.dtype.itemsize
    nb = 1
    for cand in (4, 2):
        if B % cand == 0 and 4 * cand * slab_bytes <= 52 * 1024 * 1024:
            nb = cand
            break

    w1t = w1.T
    b1t = b1.reshape(hidden, 1)
    w2t = w2.T
    b2t = b2.reshape(C, 1)

    out3d = pl.pallas_call(
        functools.partial(_se_block_kernel, nb=nb, inv_hw=1.0 / HW),
        out_shape=jax.ShapeDtypeStruct((B, C, HW), x3d.dtype),
        grid=(B // nb,),
        in_specs=[
            pl.BlockSpec((nb, C, HW), lambda b: (b, 0, 0)),
            pl.BlockSpec((hidden, C), lambda b: (0, 0)),
            pl.BlockSpec((hidden, 1), lambda b: (0, 0)),
            pl.BlockSpec((C, hidden), lambda b: (0, 0)),
            pl.BlockSpec((C, 1), lambda b: (0, 0)),
        ],
        out_specs=pl.BlockSpec((nb, C, HW), lambda b: (b, 0, 0)),
        compiler_params=pltpu.CompilerParams(
            dimension_semantics=("parallel",),
            vmem_limit_bytes=56 * 1024 * 1024,
        ),
    )(x3d, w1t, b1t, w2t, b2t)

    return out3d.reshape(B, C, H, W)


# speedup vs baseline: 1.0019x; 1.0019x over previous
import functools

import jax
import jax.numpy as jnp
from jax.experimental import pallas as pl
from jax.experimental.pallas import tpu as pltpu


def _se_block_kernel(x_ref, w1t_ref, b1t_ref, w2t_ref, b2t_ref, o_ref, *,
                     nb, inv_hw):
    w1t = w1t_ref[...]
    w2t = w2t_ref[...]
    b1t = b1t_ref[...]
    b2t = b2t_ref[...]
    for i in range(nb):
        x = x_ref[i]
        y = jnp.sum(x, axis=1, keepdims=True) * inv_hw
        h = jax.lax.dot_general(
            w1t, y,
            dimension_numbers=(((1,), (0,)), ((), ())),
            preferred_element_type=jnp.float32,
        ) + b1t
        h = jnp.maximum(h, 0.0)
        z = jax.lax.dot_general(
            w2t, h,
            dimension_numbers=(((1,), (0,)), ((), ())),
            preferred_element_type=jnp.float32,
        ) + b2t
        s = jax.nn.sigmoid(z)
        o_ref[i] = x * s.astype(x.dtype)


def kernel(x, w1, b1, w2, b2):
    B, C, H, W = x.shape
    HW = H * W
    hidden = w1.shape[1]
    x3d = x.reshape(B, C, HW)

    slab_bytes = C * HW * x.dtype.itemsize
    nb = 1
    for cand in (4, 2):
        if B % cand == 0 and 4 * cand * slab_bytes <= 52 * 1024 * 1024:
            nb = cand
            break

    w1t = w1.T
    b1t = b1.reshape(hidden, 1)
    w2t = w2.T
    b2t = b2.reshape(C, 1)

    out3d = pl.pallas_call(
        functools.partial(_se_block_kernel, nb=nb, inv_hw=1.0 / HW),
        out_shape=jax.ShapeDtypeStruct((B, C, HW), x3d.dtype),
        grid=(B // nb,),
        in_specs=[
            pl.BlockSpec((nb, C, HW), lambda b: (b, 0, 0)),
            pl.BlockSpec((hidden, C), lambda b: (0, 0)),
            pl.BlockSpec((hidden, 1), lambda b: (0, 0)),
            pl.BlockSpec((C, hidden), lambda b: (0, 0)),
            pl.BlockSpec((C, 1), lambda b: (0, 0)),
        ],
        out_specs=pl.BlockSpec((nb, C, HW), lambda b: (b, 0, 0)),
        compiler_params=pltpu.CompilerParams(
            dimension_semantics=("parallel",),
            vmem_limit_bytes=56 * 1024 * 1024,
        ),
    )(x3d, w1t, b1t, w2t, b2t)

    return out3d.reshape(B, C, H, W)


# speedup vs baseline: 1.0033x; 1.0015x over previous
import functools

import jax
import jax.numpy as jnp
from jax.experimental import pallas as pl
from jax.experimental.pallas import tpu as pltpu


def _se_block_kernel(x_ref, w1t_ref, b1t_ref, w2t_ref, b2t_ref, o_ref, *,
                     nb, inv_hw):
    w1t = w1t_ref[...]
    w2t = w2t_ref[...]
    b1t = b1t_ref[...]
    b2t = b2t_ref[...]
    y = jnp.concatenate(
        [jnp.sum(x_ref[i], axis=1, keepdims=True) for i in range(nb)],
        axis=1,
    ) * inv_hw
    h = jax.lax.dot_general(
        w1t, y,
        dimension_numbers=(((1,), (0,)), ((), ())),
        preferred_element_type=jnp.float32,
    ) + b1t
    h = jnp.maximum(h, 0.0)
    z = jax.lax.dot_general(
        w2t, h,
        dimension_numbers=(((1,), (0,)), ((), ())),
        preferred_element_type=jnp.float32,
    ) + b2t
    s = jax.nn.sigmoid(z)
    for i in range(nb):
        o_ref[i] = x_ref[i] * s[:, i:i + 1]


def kernel(x, w1, b1, w2, b2):
    B, C, H, W = x.shape
    HW = H * W
    hidden = w1.shape[1]
    x3d = x.reshape(B, C, HW)

    slab_bytes = C * HW * x.dtype.itemsize
    nb = 1
    for cand in (4, 2):
        if B % cand == 0 and 4 * cand * slab_bytes <= 52 * 1024 * 1024:
            nb = cand
            break

    w1t = w1.T
    b1t = b1.reshape(hidden, 1)
    w2t = w2.T
    b2t = b2.reshape(C, 1)

    out3d = pl.pallas_call(
        functools.partial(_se_block_kernel, nb=nb, inv_hw=1.0 / HW),
        out_shape=jax.ShapeDtypeStruct((B, C, HW), x3d.dtype),
        grid=(B // nb,),
        in_specs=[
            pl.BlockSpec((nb, C, HW), lambda b: (b, 0, 0)),
            pl.BlockSpec((hidden, C), lambda b: (0, 0)),
            pl.BlockSpec((hidden, 1), lambda b: (0, 0)),
            pl.BlockSpec((C, hidden), lambda b: (0, 0)),
            pl.BlockSpec((C, 1), lambda b: (0, 0)),
        ],
        out_specs=pl.BlockSpec((nb, C, HW), lambda b: (b, 0, 0)),
        compiler_params=pltpu.CompilerParams(
            dimension_semantics=("parallel",),
            vmem_limit_bytes=56 * 1024 * 1024,
        ),
    )(x3d, w1t, b1t, w2t, b2t)

    return out3d.reshape(B, C, H, W)
